# baseline (speedup 1.0000x reference)
import functools

import jax
import jax.numpy as jnp
from jax import lax
from jax.experimental import pallas as pl
from jax.experimental.pallas import tpu as pltpu
from jax.experimental.pallas import tpu_sc as plsc

N = 10000
D = 128
NC = 2
NS = 16
NT = NC * NS

EPT = 10240
EPAD = NT * EPT
EBP = EPT // 128

EBd = 80
NP = 10240
RPTd = NP // NS

HALF = 5120
NDUMP = 128
NPa = HALF + NDUMP
RPT = NPa // NS

_mesh = plsc.VectorSubcoreMesh(core_axis_name="c", subcore_axis_name="s")


@functools.partial(
    pl.kernel,
    out_type=(jax.ShapeDtypeStruct((NC, NT, EPT), jnp.int32),
              jax.ShapeDtypeStruct((NC, NT, EPT), jnp.int32),
              jax.ShapeDtypeStruct((NC, NT, 16), jnp.int32)),
    mesh=_mesh,
    compiler_params=pltpu.CompilerParams(needs_layout_passes=False),
    scratch_types=[
        pltpu.VMEM((EPT,), jnp.int32),
        pltpu.VMEM((EPT,), jnp.int32),
        pltpu.VMEM((EPT + 16,), jnp.int32),
        pltpu.VMEM((EPT + 16,), jnp.int32),
        pltpu.VMEM((EPT + 16,), jnp.int32),
        pltpu.VMEM((EPT + 16,), jnp.int32),
        pltpu.VMEM((16,), jnp.int32),
    ],
)
def _partition_kernel(src_hbm, dst_hbm, zi_hbm, dump_hbm,
                      psrc_hbm, pdst_hbm, cnt_hbm,
                      srcv, dstv, os0, od0, os1, od1, cntv):
    c = lax.axis_index("c")
    s = lax.axis_index("s")
    g = c * NS + s
    pltpu.sync_copy(src_hbm.at[pl.ds(g * EPT, EPT)], srcv)
    pltpu.sync_copy(dst_hbm.at[pl.ds(g * EPT, EPT)], dstv)
    pltpu.sync_copy(zi_hbm, os0.at[pl.ds(0, EPT)])
    pltpu.sync_copy(zi_hbm, os1.at[pl.ds(0, EPT)])
    pltpu.sync_copy(dump_hbm, od0.at[pl.ds(0, EPT)])
    pltpu.sync_copy(dump_hbm, od1.at[pl.ds(0, EPT)])

    iota = lax.iota(jnp.int32, 16)
    TRASH = EPT

    def _iscan(x):
        for sh in (1, 2, 4, 8):
            g = lax.gather(
                x, jnp.maximum(iota - sh, 0)[:, None],
                lax.GatherDimensionNumbers(
                    offset_dims=(), collapsed_slice_dims=(0,),
                    start_index_map=(0,)),
                (1,), mode=lax.GatherScatterMode.PROMISE_IN_BOUNDS)
            x = x + jnp.where(iota >= sh, g, 0)
        return x

    def body(j, offs):
        off0, off1 = offs
        sv = srcv[pl.ds(j * 16, 16)]
        dv = dstv[pl.ds(j * 16, 16)]
        m0 = (dv >= 0) & (dv < HALF)
        m1 = dv >= HALF
        i0 = m0.astype(jnp.int32)
        i1 = m1.astype(jnp.int32)
        incl0 = _iscan(i0)
        incl1 = _iscan(i1)
        t0 = jnp.where(m0, off0 + incl0 - i0, TRASH)
        t1 = jnp.where(m1, off1 + incl1 - i1, TRASH)
        plsc.store_scatter(os0, [t0], sv)
        plsc.store_scatter(od0, [t0], dv)
        plsc.store_scatter(os1, [t1], sv)
        plsc.store_scatter(od1, [t1], dv - HALF)
        return off0 + incl0[15], off1 + incl1[15]

    off0, off1 = lax.fori_loop(0, EPT // 16, body,
                               (jnp.int32(0), jnp.int32(0)))
    pltpu.sync_copy(os0.at[pl.ds(0, EPT)], psrc_hbm.at[0].at[g])
    pltpu.sync_copy(od0.at[pl.ds(0, EPT)], pdst_hbm.at[0].at[g])
    pltpu.sync_copy(os1.at[pl.ds(0, EPT)], psrc_hbm.at[1].at[g])
    pltpu.sync_copy(od1.at[pl.ds(0, EPT)], pdst_hbm.at[1].at[g])
    cntv[pl.ds(0, 16)] = jnp.broadcast_to((off0 + 127) // 128, (16,)).astype(jnp.int32)
    pltpu.sync_copy(cntv, cnt_hbm.at[0].at[g])
    cntv[pl.ds(0, 16)] = jnp.broadcast_to((off1 + 127) // 128, (16,)).astype(jnp.int32)
    pltpu.sync_copy(cntv, cnt_hbm.at[1].at[g])


@functools.partial(
    pl.kernel,
    out_type=jax.ShapeDtypeStruct((NC, NP), jnp.float32),
    mesh=_mesh,
    scratch_types=[
        pltpu.VMEM((EBd, 128), jnp.int32),
        pltpu.VMEM((128,), jnp.float32),
        pltpu.VMEM((RPTd,), jnp.float32),
        pltpu.VMEM_SHARED((NP,), jnp.float32),
    ],
)
def _degree_kernel(dst_hbm, deg_hbm, dstv, onesv, zv, acc):
    c = lax.axis_index("c")
    s = lax.axis_index("s")
    g = c * NS + s
    for k in range(8):
        onesv[pl.ds(k * 16, 16)] = jnp.ones((16,), jnp.float32)
    for k in range(RPTd // 16):
        zv[pl.ds(k * 16, 16)] = jnp.zeros((16,), jnp.float32)
    pltpu.sync_copy(zv, acc.at[pl.ds(s * RPTd, RPTd)])
    pltpu.sync_copy(dst_hbm.at[pl.ds(g * EBd, EBd)], dstv)
    plsc.subcore_barrier()
    for j in range(EBd):
        pltpu.sync_copy(onesv, acc.at[dstv.at[j]], add=True)
    plsc.subcore_barrier()
    pltpu.sync_copy(acc.at[pl.ds(s * RPTd, RPTd)],
                    deg_hbm.at[c, pl.ds(s * RPTd, RPTd)])


@functools.partial(
    pl.kernel,
    out_type=jax.ShapeDtypeStruct((NC, NPa, D), jnp.float32),
    mesh=_mesh,
    scratch_types=[
        pltpu.VMEM((EBP, 128), jnp.int32),
        pltpu.VMEM((EBP, 128), jnp.int32),
        pltpu.VMEM((16,), jnp.int32),
        pltpu.VMEM((128, D), jnp.float32),
        pltpu.VMEM((128, D), jnp.float32),
        pltpu.VMEM_SHARED((NPa, D), jnp.float32),
        pltpu.SemaphoreType.DMA,
        pltpu.SemaphoreType.DMA,
    ],
)
def _agg_kernel(h_hbm, psrc_hbm, pdst_hbm, cnt_hbm, zeros_hbm, out_hbm,
                srcv, dstv, cntv, buf0, buf1, acc, gsem, ssem):
    c = lax.axis_index("c")
    s = lax.axis_index("s")
    pltpu.sync_copy(zeros_hbm, acc.at[pl.ds(s * RPT, RPT)])
    plsc.subcore_barrier()

    def run_list(gl):
        pltpu.sync_copy(psrc_hbm.at[c].at[gl], srcv)
        pltpu.sync_copy(pdst_hbm.at[c].at[gl], dstv)
        pltpu.sync_copy(cnt_hbm.at[c].at[gl], cntv)
        nch = cntv[pl.ds(0, 16)][0]

        def wait_gather(buf):
            pltpu.make_async_copy(h_hbm.at[srcv.at[0]], buf, gsem).wait()

        def drain_scatter(buf):
            pltpu.make_async_copy(buf, acc.at[dstv.at[0]], ssem).wait()

        @pl.when(nch > 0)
        def _():
            pltpu.async_copy(h_hbm.at[srcv.at[0]], buf0, gsem)

            def body(j, carry):
                even = lax.rem(j, 2) == 0

                def step(cur, nxt):
                    @pl.when(j > 0)
                    def _():
                        drain_scatter(nxt)

                    @pl.when(j + 1 < nch)
                    def _():
                        pltpu.async_copy(h_hbm.at[srcv.at[j + 1]], nxt, gsem)

                    wait_gather(cur)
                    pltpu.async_copy(cur, acc.at[dstv.at[j]], ssem, add=True)

                @pl.when(even)
                def _():
                    step(buf0, buf1)

                @pl.when(jnp.logical_not(even))
                def _():
                    step(buf1, buf0)

                return carry

            lax.fori_loop(0, nch, body, jnp.int32(0))
            drain_scatter(buf0)

    run_list(s)
    run_list(s + NS)
    plsc.subcore_barrier()
    pltpu.sync_copy(acc.at[pl.ds(s * RPT, RPT)],
                    out_hbm.at[c].at[pl.ds(s * RPT, RPT)])


R = 1000


def _mm_first_body(x_ref, w_ref, d0_ref, d1_ref, hp_ref, dinv_ref):
    dinv = lax.rsqrt(d0_ref[...] + d1_ref[...] + 1.0)
    hp_ref[...] = jnp.dot(x_ref[...], w_ref[...],
                          preferred_element_type=jnp.float32) * dinv
    dinv_ref[...] = dinv


def _mm_mid_body(p_ref, hp_ref, dinv_ref, b_ref, w_ref, out_ref):
    dinv = dinv_ref[...]
    t = (p_ref[...] + hp_ref[...]) * dinv + b_ref[...]
    t = jnp.maximum(t, 0.0)
    out_ref[...] = jnp.dot(t, w_ref[...],
                           preferred_element_type=jnp.float32) * dinv


def _final_body(p_ref, hp_ref, dinv_ref, b_ref, out_ref):
    z = (p_ref[...] + hp_ref[...]) * dinv_ref[...] + b_ref[...]
    m = jnp.max(z, axis=1, keepdims=True)
    lse = jnp.log(jnp.sum(jnp.exp(z - m), axis=1, keepdims=True))
    out_ref[...] = z - m - lse


def _row_spec(width):
    return pl.BlockSpec((R, width), lambda i: (i, 0))


def _full_spec(shape):
    return pl.BlockSpec(shape, lambda i: tuple(0 for _ in shape))


def _mm_first(x, w, d0, d1):
    return pl.pallas_call(
        _mm_first_body,
        grid=(N // R,),
        in_specs=[_row_spec(D), _full_spec((D, D)), _row_spec(1), _row_spec(1)],
        out_specs=[_row_spec(D), _row_spec(1)],
        out_shape=[jax.ShapeDtypeStruct((N, D), jnp.float32),
                   jax.ShapeDtypeStruct((N, 1), jnp.float32)],
    )(x, w, d0, d1)


def _mm_mid(p, hp, dinv, b, w):
    return pl.pallas_call(
        _mm_mid_body,
        grid=(N // R,),
        in_specs=[_row_spec(D), _row_spec(D), _row_spec(1),
                  _full_spec((1, D)), _full_spec((D, D))],
        out_specs=_row_spec(D),
        out_shape=jax.ShapeDtypeStruct((N, D), jnp.float32),
    )(p, hp, dinv, b, w)


def _final(p, hp, dinv, b):
    return pl.pallas_call(
        _final_body,
        grid=(N // R,),
        in_specs=[_row_spec(D), _row_spec(D), _row_spec(1),
                  _full_spec((1, D))],
        out_specs=_row_spec(D),
        out_shape=jax.ShapeDtypeStruct((N, D), jnp.float32),
    )(p, hp, dinv, b)


def kernel(x, edge_index, W1, b1, W2, b2, W3, b3):
    src = edge_index[0]
    dst = edge_index[1]
    pad = EPAD - src.shape[0]
    src1d = jnp.concatenate([src, jnp.zeros((pad,), jnp.int32)])
    dst1d = jnp.concatenate([dst, jnp.full((pad,), -1, jnp.int32)])
    dst2d_deg = jnp.where(dst1d < 0, N, dst1d).reshape(NT * EBd, 128)
    zi = jnp.zeros((EPT,), jnp.int32)
    dump = HALF + (jnp.arange(EPT, dtype=jnp.int32) % NDUMP)
    zeros = jnp.zeros((RPT, D), jnp.float32)

    psrc, pdst, cnt = _partition_kernel(src1d, dst1d, zi, dump)
    psrc = psrc.reshape(NC, NT, EBP, 128)
    pdst = pdst.reshape(NC, NT, EBP, 128)

    degp = _degree_kernel(dst2d_deg)
    d0 = degp[0].reshape(NP, 1)
    d1 = degp[1].reshape(NP, 1)

    h1p, dinv = _mm_first(x, W1, d0[:N], d1[:N])
    P = _agg_kernel(h1p, psrc, pdst, cnt, zeros)
    S = jnp.concatenate([P[0, :HALF], P[1, :N - HALF]])
    h2p = _mm_mid(S, h1p, dinv, b1.reshape(1, D), W2)
    P = _agg_kernel(h2p, psrc, pdst, cnt, zeros)
    S = jnp.concatenate([P[0, :HALF], P[1, :N - HALF]])
    h3p = _mm_mid(S, h2p, dinv, b2.reshape(1, D), W3)
    P = _agg_kernel(h3p, psrc, pdst, cnt, zeros)
    S = jnp.concatenate([P[0, :HALF], P[1, :N - HALF]])
    return _final(S, h3p, dinv, b3.reshape(1, D))


# speedup vs baseline: 1.0013x; 1.0013x over previous
import functools

import jax
import jax.numpy as jnp
from jax import lax
from jax.experimental import pallas as pl
from jax.experimental.pallas import tpu as pltpu
from jax.experimental.pallas import tpu_sc as plsc

N = 10000
D = 128
NC = 2
NS = 16
NT = NC * NS

EPT = 10240
EPAD = NT * EPT
EBP = EPT // 128

EBd = 80
NP = 10240
RPTd = NP // NS

HALF = 5120
NDUMP = 128
NPa = HALF + NDUMP
RPT = NPa // NS

_mesh = plsc.VectorSubcoreMesh(core_axis_name="c", subcore_axis_name="s")


@functools.partial(
    pl.kernel,
    out_type=(jax.ShapeDtypeStruct((NC, NT, EPT), jnp.int32),
              jax.ShapeDtypeStruct((NC, NT, EPT), jnp.int32),
              jax.ShapeDtypeStruct((NC, NT, 16), jnp.int32),
              jax.ShapeDtypeStruct((NC, NP), jnp.float32)),
    mesh=_mesh,
    compiler_params=pltpu.CompilerParams(needs_layout_passes=False),
    scratch_types=[
        pltpu.VMEM((EPT,), jnp.int32),
        pltpu.VMEM((EPT,), jnp.int32),
        pltpu.VMEM((EPT + 16,), jnp.int32),
        pltpu.VMEM((EPT + 16,), jnp.int32),
        pltpu.VMEM((EPT + 16,), jnp.int32),
        pltpu.VMEM((EPT + 16,), jnp.int32),
        pltpu.VMEM((16,), jnp.int32),
        pltpu.VMEM((EBd, 128), jnp.int32),
        pltpu.VMEM((128,), jnp.float32),
        pltpu.VMEM((RPTd,), jnp.float32),
        pltpu.VMEM_SHARED((NP,), jnp.float32),
    ],
)
def _partition_kernel(src_hbm, dst_hbm, dstdeg_hbm, zi_hbm, dump_hbm,
                      psrc_hbm, pdst_hbm, cnt_hbm, deg_hbm,
                      srcv, dstv, os0, od0, os1, od1, cntv,
                      dstv2, onesv, zv, dacc):
    c = lax.axis_index("c")
    s = lax.axis_index("s")
    g = c * NS + s
    pltpu.sync_copy(src_hbm.at[pl.ds(g * EPT, EPT)], srcv)
    pltpu.sync_copy(dst_hbm.at[pl.ds(g * EPT, EPT)], dstv)
    pltpu.sync_copy(zi_hbm, os0.at[pl.ds(0, EPT)])
    pltpu.sync_copy(zi_hbm, os1.at[pl.ds(0, EPT)])
    pltpu.sync_copy(dump_hbm, od0.at[pl.ds(0, EPT)])
    pltpu.sync_copy(dump_hbm, od1.at[pl.ds(0, EPT)])

    for k in range(8):
        onesv[pl.ds(k * 16, 16)] = jnp.ones((16,), jnp.float32)
    for k in range(RPTd // 16):
        zv[pl.ds(k * 16, 16)] = jnp.zeros((16,), jnp.float32)
    pltpu.sync_copy(zv, dacc.at[pl.ds(s * RPTd, RPTd)])
    pltpu.sync_copy(dstdeg_hbm.at[pl.ds(g * EBd, EBd)], dstv2)
    plsc.subcore_barrier()
    for j in range(EBd):
        pltpu.sync_copy(onesv, dacc.at[dstv2.at[j]], add=True)

    iota = lax.iota(jnp.int32, 16)
    TRASH = EPT

    def _iscan(x):
        for sh in (1, 2, 4, 8):
            g = lax.gather(
                x, jnp.maximum(iota - sh, 0)[:, None],
                lax.GatherDimensionNumbers(
                    offset_dims=(), collapsed_slice_dims=(0,),
                    start_index_map=(0,)),
                (1,), mode=lax.GatherScatterMode.PROMISE_IN_BOUNDS)
            x = x + jnp.where(iota >= sh, g, 0)
        return x

    def body(j, offs):
        off0, off1 = offs
        sv = srcv[pl.ds(j * 16, 16)]
        dv = dstv[pl.ds(j * 16, 16)]
        m0 = (dv >= 0) & (dv < HALF)
        m1 = dv >= HALF
        i0 = m0.astype(jnp.int32)
        i1 = m1.astype(jnp.int32)
        incl0 = _iscan(i0)
        incl1 = _iscan(i1)
        t0 = jnp.where(m0, off0 + incl0 - i0, TRASH)
        t1 = jnp.where(m1, off1 + incl1 - i1, TRASH)
        plsc.store_scatter(os0, [t0], sv)
        plsc.store_scatter(od0, [t0], dv)
        plsc.store_scatter(os1, [t1], sv)
        plsc.store_scatter(od1, [t1], dv - HALF)
        return off0 + incl0[15], off1 + incl1[15]

    off0, off1 = lax.fori_loop(0, EPT // 16, body,
                               (jnp.int32(0), jnp.int32(0)))
    plsc.subcore_barrier()
    pltpu.sync_copy(dacc.at[pl.ds(s * RPTd, RPTd)],
                    deg_hbm.at[c, pl.ds(s * RPTd, RPTd)])
    pltpu.sync_copy(os0.at[pl.ds(0, EPT)], psrc_hbm.at[0].at[g])
    pltpu.sync_copy(od0.at[pl.ds(0, EPT)], pdst_hbm.at[0].at[g])
    pltpu.sync_copy(os1.at[pl.ds(0, EPT)], psrc_hbm.at[1].at[g])
    pltpu.sync_copy(od1.at[pl.ds(0, EPT)], pdst_hbm.at[1].at[g])
    cntv[pl.ds(0, 16)] = jnp.broadcast_to((off0 + 127) // 128, (16,)).astype(jnp.int32)
    pltpu.sync_copy(cntv, cnt_hbm.at[0].at[g])
    cntv[pl.ds(0, 16)] = jnp.broadcast_to((off1 + 127) // 128, (16,)).astype(jnp.int32)
    pltpu.sync_copy(cntv, cnt_hbm.at[1].at[g])


@functools.partial(
    pl.kernel,
    out_type=jax.ShapeDtypeStruct((NC, NPa, D), jnp.float32),
    mesh=_mesh,
    scratch_types=[
        pltpu.VMEM((EBP, 128), jnp.int32),
        pltpu.VMEM((EBP, 128), jnp.int32),
        pltpu.VMEM((16,), jnp.int32),
        pltpu.VMEM((128, D), jnp.float32),
        pltpu.VMEM((128, D), jnp.float32),
        pltpu.VMEM_SHARED((NPa, D), jnp.float32),
        pltpu.SemaphoreType.DMA,
        pltpu.SemaphoreType.DMA,
    ],
)
def _agg_kernel(h_hbm, psrc_hbm, pdst_hbm, cnt_hbm, zeros_hbm, out_hbm,
                srcv, dstv, cntv, buf0, buf1, acc, gsem, ssem):
    c = lax.axis_index("c")
    s = lax.axis_index("s")
    pltpu.sync_copy(zeros_hbm, acc.at[pl.ds(s * RPT, RPT)])
    plsc.subcore_barrier()

    def run_list(gl):
        pltpu.sync_copy(psrc_hbm.at[c].at[gl], srcv)
        pltpu.sync_copy(pdst_hbm.at[c].at[gl], dstv)
        pltpu.sync_copy(cnt_hbm.at[c].at[gl], cntv)
        nch = cntv[pl.ds(0, 16)][0]

        def wait_gather(buf):
            pltpu.make_async_copy(h_hbm.at[srcv.at[0]], buf, gsem).wait()

        def drain_scatter(buf):
            pltpu.make_async_copy(buf, acc.at[dstv.at[0]], ssem).wait()

        @pl.when(nch > 0)
        def _():
            pltpu.async_copy(h_hbm.at[srcv.at[0]], buf0, gsem)

            def body(j, carry):
                even = lax.rem(j, 2) == 0

                def step(cur, nxt):
                    @pl.when(j > 0)
                    def _():
                        drain_scatter(nxt)

                    @pl.when(j + 1 < nch)
                    def _():
                        pltpu.async_copy(h_hbm.at[srcv.at[j + 1]], nxt, gsem)

                    wait_gather(cur)
                    pltpu.async_copy(cur, acc.at[dstv.at[j]], ssem, add=True)

                @pl.when(even)
                def _():
                    step(buf0, buf1)

                @pl.when(jnp.logical_not(even))
                def _():
                    step(buf1, buf0)

                return carry

            lax.fori_loop(0, nch, body, jnp.int32(0))
            drain_scatter(buf0)

    run_list(s)
    run_list(s + NS)
    plsc.subcore_barrier()
    pltpu.sync_copy(acc.at[pl.ds(s * RPT, RPT)],
                    out_hbm.at[c].at[pl.ds(s * RPT, RPT)])


R = 1000


def _mm_first_body(x_ref, w_ref, d0_ref, d1_ref, hp_ref, dinv_ref):
    dinv = lax.rsqrt(d0_ref[...] + d1_ref[...] + 1.0)
    hp_ref[...] = jnp.dot(x_ref[...], w_ref[...],
                          preferred_element_type=jnp.float32) * dinv
    dinv_ref[...] = dinv


def _mm_mid_body(p_ref, hp_ref, dinv_ref, b_ref, w_ref, out_ref):
    dinv = dinv_ref[...]
    t = (p_ref[...] + hp_ref[...]) * dinv + b_ref[...]
    t = jnp.maximum(t, 0.0)
    out_ref[...] = jnp.dot(t, w_ref[...],
                           preferred_element_type=jnp.float32) * dinv


def _final_body(p_ref, hp_ref, dinv_ref, b_ref, out_ref):
    z = (p_ref[...] + hp_ref[...]) * dinv_ref[...] + b_ref[...]
    m = jnp.max(z, axis=1, keepdims=True)
    lse = jnp.log(jnp.sum(jnp.exp(z - m), axis=1, keepdims=True))
    out_ref[...] = z - m - lse


def _row_spec(width):
    return pl.BlockSpec((R, width), lambda i: (i, 0))


def _full_spec(shape):
    return pl.BlockSpec(shape, lambda i: tuple(0 for _ in shape))


def _mm_first(x, w, d0, d1):
    return pl.pallas_call(
        _mm_first_body,
        grid=(N // R,),
        in_specs=[_row_spec(D), _full_spec((D, D)), _row_spec(1), _row_spec(1)],
        out_specs=[_row_spec(D), _row_spec(1)],
        out_shape=[jax.ShapeDtypeStruct((N, D), jnp.float32),
                   jax.ShapeDtypeStruct((N, 1), jnp.float32)],
    )(x, w, d0, d1)


def _mm_mid(p, hp, dinv, b, w):
    return pl.pallas_call(
        _mm_mid_body,
        grid=(N // R,),
        in_specs=[_row_spec(D), _row_spec(D), _row_spec(1),
                  _full_spec((1, D)), _full_spec((D, D))],
        out_specs=_row_spec(D),
        out_shape=jax.ShapeDtypeStruct((N, D), jnp.float32),
    )(p, hp, dinv, b, w)


def _final(p, hp, dinv, b):
    return pl.pallas_call(
        _final_body,
        grid=(N // R,),
        in_specs=[_row_spec(D), _row_spec(D), _row_spec(1),
                  _full_spec((1, D))],
        out_specs=_row_spec(D),
        out_shape=jax.ShapeDtypeStruct((N, D), jnp.float32),
    )(p, hp, dinv, b)


def kernel(x, edge_index, W1, b1, W2, b2, W3, b3):
    src = edge_index[0]
    dst = edge_index[1]
    pad = EPAD - src.shape[0]
    src1d = jnp.concatenate([src, jnp.zeros((pad,), jnp.int32)])
    dst1d = jnp.concatenate([dst, jnp.full((pad,), -1, jnp.int32)])
    dst2d_deg = jnp.where(dst1d < 0, N, dst1d).reshape(NT * EBd, 128)
    zi = jnp.zeros((EPT,), jnp.int32)
    dump = HALF + (jnp.arange(EPT, dtype=jnp.int32) % NDUMP)
    zeros = jnp.zeros((RPT, D), jnp.float32)

    psrc, pdst, cnt, degp = _partition_kernel(src1d, dst1d, dst2d_deg,
                                              zi, dump)
    psrc = psrc.reshape(NC, NT, EBP, 128)
    pdst = pdst.reshape(NC, NT, EBP, 128)

    d0 = degp[0].reshape(NP, 1)
    d1 = degp[1].reshape(NP, 1)

    h1p, dinv = _mm_first(x, W1, d0[:N], d1[:N])
    P = _agg_kernel(h1p, psrc, pdst, cnt, zeros)
    S = jnp.concatenate([P[0, :HALF], P[1, :N - HALF]])
    h2p = _mm_mid(S, h1p, dinv, b1.reshape(1, D), W2)
    P = _agg_kernel(h2p, psrc, pdst, cnt, zeros)
    S = jnp.concatenate([P[0, :HALF], P[1, :N - HALF]])
    h3p = _mm_mid(S, h2p, dinv, b2.reshape(1, D), W3)
    P = _agg_kernel(h3p, psrc, pdst, cnt, zeros)
    S = jnp.concatenate([P[0, :HALF], P[1, :N - HALF]])
    return _final(S, h3p, dinv, b3.reshape(1, D))
